# Initial kernel scaffold; baseline (speedup 1.0000x reference)
import jax
import jax.numpy as jnp
from jax.experimental import pallas as pl


def kernel(tensor, boost_tensor, boost_percent):
    raise NotImplementedError("write your pallas kernel here")


# speedup vs baseline: 53.8174x; 53.8174x over previous
import math

import jax
import jax.numpy as jnp
from jax.experimental import pallas as pl
from jax.experimental.pallas import tpu as pltpu


def _sortable_i32(x):
    bits = jax.lax.bitcast_convert_type(x, jnp.int32)
    return jnp.where(bits >= 0, bits, bits ^ jnp.int32(0x7FFFFFFF))


def _kth_largest_key(key, k):
    B = key.shape[0]
    cnt = jnp.sum((key >= 0).astype(jnp.int32), axis=1, keepdims=True)
    p = jnp.where(cnt >= k, jnp.int32(0), jnp.int32(-(2**31)))
    for b in range(30, -1, -1):
        cand = p + jnp.int32(1 << b)
        cnt = jnp.sum((key >= cand).astype(jnp.int32), axis=1, keepdims=True)
        p = jnp.where(cnt >= k, cand, p)
    return p


def _kwinners_body(max_active, min_active, x_ref, bt_ref, bp_ref, out_ref):
    x = x_ref[...]
    btin = bt_ref[...]
    bp = bp_ref[0]

    row_max = jnp.max(x, axis=1, keepdims=True)
    safe_max = jnp.where(row_max == 0.0, jnp.ones_like(row_max), row_max)
    bt = btin + bp * (x / safe_max)
    boosted = jnp.maximum(x, 0.0) + bt

    key = _sortable_i32(boosted)
    p = _kth_largest_key(key, max_active)
    t = jnp.where((key >= p) & (x > 0.0), 1.0, 0.0).astype(jnp.float32)
    out_ref[...] = t

    total = jnp.sum(t)

    @pl.when(total < jnp.float32(min_active))
    def _rescue():
        j = jnp.ceil(jnp.float32(min_active) - total).astype(jnp.int32)
        bkey = _sortable_i32(bt)
        p2 = _kth_largest_key(bkey, j)
        resc = (bkey >= p2).astype(jnp.float32)
        out_ref[...] = jnp.maximum(t, resc)


def kernel(tensor, boost_tensor, boost_percent):
    B, N = tensor.shape
    max_active = int(math.ceil(0.02 * N))
    min_active = int(math.floor(0.002 * N))
    bp = jnp.asarray(boost_percent, jnp.float32).reshape((1,))

    body = lambda x_ref, bt_ref, bp_ref, out_ref: _kwinners_body(
        max_active, min_active, x_ref, bt_ref, bp_ref, out_ref
    )
    return pl.pallas_call(
        body,
        out_shape=jax.ShapeDtypeStruct((B, N), jnp.float32),
        in_specs=[
            pl.BlockSpec(memory_space=pltpu.VMEM),
            pl.BlockSpec(memory_space=pltpu.VMEM),
            pl.BlockSpec(memory_space=pltpu.SMEM),
        ],
        out_specs=pl.BlockSpec(memory_space=pltpu.VMEM),
    )(tensor, boost_tensor, bp)
